# baseline (speedup 1.0000x reference)
import jax
import jax.numpy as jnp
from jax.experimental import pallas as pl
from jax.experimental.pallas import tpu as pltpu


def _log_sigmoid(x):
    return jnp.minimum(x, 0.0) - jnp.log1p(jnp.exp(-jnp.abs(x)))


def _body(c_ref, o_ref, n_ref, out_ref):
    c = c_ref[...]
    pos = jnp.sum(o_ref[...] * c, axis=0)
    acc = _log_sigmoid(pos)
    nneg = n_ref.shape[0]
    for k in range(nneg):
        s = jnp.sum(n_ref[k] * c, axis=0)
        acc = acc + _log_sigmoid(-s)
    partial = jnp.sum(acc)

    @pl.when(pl.program_id(0) == 0)
    def _():
        out_ref[0, 0] = 0.0

    out_ref[0, 0] += partial


def kernel(center_word_vec, outside_word_vec, neg_word_vec):
    size, dim = center_word_vec.shape
    nneg = neg_word_vec.shape[1]
    c_t = center_word_vec.T
    o_t = outside_word_vec.T
    n_t = jnp.transpose(neg_word_vec, (1, 2, 0))
    B = 1024
    grid = size // B
    out = pl.pallas_call(
        _body,
        grid=(grid,),
        in_specs=[
            pl.BlockSpec((dim, B), lambda i: (0, i)),
            pl.BlockSpec((dim, B), lambda i: (0, i)),
            pl.BlockSpec((nneg, dim, B), lambda i: (0, 0, i)),
        ],
        out_specs=pl.BlockSpec(memory_space=pltpu.SMEM),
        out_shape=jax.ShapeDtypeStruct((1, 1), jnp.float32),
    )(c_t, o_t, n_t)
    return -(out[0, 0] / size)


# speedup vs baseline: 1.4038x; 1.4038x over previous
import jax
import jax.numpy as jnp
from jax.experimental import pallas as pl
from jax.experimental.pallas import tpu as pltpu


def _log_sigmoid(x):
    return jnp.minimum(x, 0.0) - jnp.log1p(jnp.exp(-jnp.abs(x)))


def _body(c_ref, o_ref, n_ref, out_ref):
    c = c_ref[...]
    pos = jnp.sum(o_ref[...] * c, axis=0)
    acc = _log_sigmoid(pos)
    nneg = n_ref.shape[0]
    for k in range(nneg):
        s = jnp.sum(n_ref[k] * c, axis=0)
        acc = acc + _log_sigmoid(-s)
    partial = jnp.sum(acc)

    @pl.when(pl.program_id(0) == 0)
    def _():
        out_ref[0, 0] = 0.0

    out_ref[0, 0] += partial


def kernel(center_word_vec, outside_word_vec, neg_word_vec):
    size, dim = center_word_vec.shape
    nneg = neg_word_vec.shape[1]
    c_t = center_word_vec.T
    o_t = outside_word_vec.T
    n_t = jnp.transpose(neg_word_vec, (1, 2, 0))
    B = 4096
    grid = size // B
    out = pl.pallas_call(
        _body,
        grid=(grid,),
        in_specs=[
            pl.BlockSpec((dim, B), lambda i: (0, i)),
            pl.BlockSpec((dim, B), lambda i: (0, i)),
            pl.BlockSpec((nneg, dim, B), lambda i: (0, 0, i)),
        ],
        out_specs=pl.BlockSpec(memory_space=pltpu.SMEM),
        out_shape=jax.ShapeDtypeStruct((1, 1), jnp.float32),
    )(c_t, o_t, n_t)
    return -(out[0, 0] / size)
